# baseline (speedup 1.0000x reference)
import functools

import jax
import jax.numpy as jnp
from jax import lax
from jax.experimental import pallas as pl
from jax.experimental.pallas import tpu as pltpu
from jax.experimental.pallas import tpu_sc as plsc

W = 5
D = 64
B = 1024
L = 200
N = B * L


def _sc_gather(idx_flat, table):
    info = plsc.get_sparse_core_info()
    nw = info.num_cores * info.num_subcores
    per_w = N // nw
    chunk = 1600
    n_chunks = per_w // chunk

    mesh = plsc.VectorSubcoreMesh(core_axis_name="c", subcore_axis_name="s")

    @functools.partial(
        pl.kernel,
        out_type=jax.ShapeDtypeStruct((N, D), jnp.float32),
        mesh=mesh,
        scratch_types=[
            pltpu.VMEM((chunk,), jnp.int32),
            pltpu.VMEM((chunk, D), jnp.float32),
            pltpu.SemaphoreType.DMA,
        ],
        compiler_params=pltpu.CompilerParams(use_tc_tiling_on_sc=False),
    )
    def gather_kernel(table_hbm, idx_hbm, out_hbm, idx_v, rows_v, sem):
        wid = lax.axis_index("s") * info.num_cores + lax.axis_index("c")

        def body(i, carry):
            base = wid * per_w + i * chunk
            pltpu.sync_copy(idx_hbm.at[pl.ds(base, chunk)], idx_v)
            pltpu.async_copy(table_hbm.at[idx_v], rows_v, sem).wait()
            pltpu.sync_copy(rows_v, out_hbm.at[pl.ds(base, chunk)])
            return carry

        lax.fori_loop(0, n_chunks, body, 0)

    return gather_kernel(table, idx_flat)


def _window_body(emb_ref, out_ref):
    e = emb_ref[...]
    bb = e.shape[0]
    parts = [e]
    for k in range(1, W):
        z = jnp.zeros((bb, k, D), jnp.float32)
        parts.append(jnp.concatenate([z, e[:, : L - k, :]], axis=1))
    out_ref[...] = jnp.concatenate(parts, axis=2)


def _tc_window(emb):
    bb = 8
    return pl.pallas_call(
        _window_body,
        grid=(B // bb,),
        in_specs=[pl.BlockSpec((bb, L, D), lambda i: (i, 0, 0))],
        out_specs=pl.BlockSpec((bb, L, W * D), lambda i: (i, 0, 0)),
        out_shape=jax.ShapeDtypeStruct((B, L, W * D), jnp.float32),
    )(emb)


def kernel(inputs, table):
    idx_flat = inputs.reshape(-1).astype(jnp.int32)
    emb = _sc_gather(idx_flat, table)
    return _tc_window(emb.reshape(B, L, D))
